# baseline (speedup 1.0000x reference)
import functools

import jax
import jax.numpy as jnp
from jax import lax
from jax.experimental import pallas as pl
from jax.experimental.pallas import tpu as pltpu
from jax.experimental.pallas import tpu_sc as plsc

_EPS = 1e-5
_L = 16


def _build_sc_call(n_tok, hidden, out_dtype):
  NW = 32
  TPW = n_tok // NW
  C = 16
  NCHUNK = TPW // C
  J16 = hidden // _L

  mesh = plsc.VectorSubcoreMesh(core_axis_name="c", subcore_axis_name="s")

  def body(ids_hbm, table_hbm, w_hbm, out_hbm,
           idx_v, w_v, rows0, rows1, ob0, ob1,
           gsem0, gsem1, osem0, osem1):
    cid = lax.axis_index("c")
    sid = lax.axis_index("s")
    wid = sid * 2 + cid
    base = wid * TPW

    pltpu.sync_copy(ids_hbm.at[pl.ds(base, TPW)], idx_v)
    pltpu.sync_copy(w_hbm, w_v)

    rows = (rows0, rows1)
    obs = (ob0, ob1)
    gsems = (gsem0, gsem1)
    osems = (osem0, osem1)

    def start_gather(g, b):
      pltpu.async_copy(table_hbm.at[idx_v.at[pl.ds(g * C, C)]], rows[b],
                       gsems[b])

    def wait_gather(b):
      pltpu.make_async_copy(table_hbm.at[idx_v.at[pl.ds(0, C)]], rows[b],
                            gsems[b]).wait()

    def compute(rv, ob):
      z = jnp.zeros((_L,), jnp.float32)

      def ss_body(j, accs):
        cbase = j * _L
        return tuple(accs[r] + rv[r, pl.ds(cbase, _L)] *
                     rv[r, pl.ds(cbase, _L)] for r in range(C))

      accs = lax.fori_loop(0, J16, ss_body, (z,) * C)

      ys = []
      for r in range(C):
        s = jnp.sum(accs[r])
        m = s * (1.0 / hidden) + _EPS
        i = lax.bitcast_convert_type(m, jnp.int32)
        i = 0x5F3759DF - lax.shift_right_arithmetic(i, 1)
        y = lax.bitcast_convert_type(i, jnp.float32)
        y = y * (1.5 - 0.5 * m * y * y)
        y = y * (1.5 - 0.5 * m * y * y)
        y = y * (1.5 - 0.5 * m * y * y)
        ys.append(y)

      def col_body(k, _):
        for jj in range(2):
          j = 2 * k + jj
          wj = w_v[pl.ds(j * _L, _L)]
          for p in range(C // 2):
            a = rv[2 * p, pl.ds(j * _L, _L)] * ys[2 * p]
            b = rv[2 * p + 1, pl.ds(j * _L, _L)] * ys[2 * p + 1]
            packed = plsc.pack(a * wj, b * wj,
                               format=plsc.PackFormat.INTERLEAVED)
            ob[p, pl.ds(j * _L, _L)] = plsc.bitcast(packed, jnp.int32)
        return 0

      lax.fori_loop(0, J16 // 2, col_body, 0)

    start_gather(0, 0)

    def chunk_body(k, carry):
      for b in (0, 1):
        g = 2 * k + b
        wait_gather(b)

        @pl.when(g + 1 < NCHUNK)
        def _():
          start_gather(g + 1, 1 - b)

        @pl.when(g >= 2)
        def _():
          pltpu.make_async_copy(obs[b].bitcast(out_dtype),
                                out_hbm.at[pl.ds(base, C)],
                                osems[b]).wait()

        compute(rows[b], obs[b])
        pltpu.async_copy(obs[b].bitcast(out_dtype),
                         out_hbm.at[pl.ds(base + g * C, C)],
                         osems[b])
      return carry

    lax.fori_loop(0, NCHUNK // 2, chunk_body, 0)
    pltpu.make_async_copy(ob0.bitcast(out_dtype),
                          out_hbm.at[pl.ds(base, C)], osem0).wait()
    pltpu.make_async_copy(ob1.bitcast(out_dtype),
                          out_hbm.at[pl.ds(base, C)], osem1).wait()

  return pl.kernel(
      body,
      out_type=jax.ShapeDtypeStruct((n_tok, hidden), out_dtype),
      mesh=mesh,
      compiler_params=pltpu.CompilerParams(needs_layout_passes=False),
      scratch_types=[
          pltpu.VMEM((TPW,), jnp.int32),
          pltpu.VMEM((hidden,), jnp.float32),
          pltpu.VMEM((C, hidden), jnp.float32),
          pltpu.VMEM((C, hidden), jnp.float32),
          pltpu.VMEM((C // 2, hidden), jnp.int32),
          pltpu.VMEM((C // 2, hidden), jnp.int32),
          pltpu.SemaphoreType.DMA,
          pltpu.SemaphoreType.DMA,
          pltpu.SemaphoreType.DMA,
          pltpu.SemaphoreType.DMA,
      ],
  )


@functools.partial(jax.jit, static_argnames=())
def kernel(input_ids, tok_emb, norm_weight):
  b, s = input_ids.shape
  vocab, hidden = tok_emb.shape
  ids = input_ids.reshape(-1).astype(jnp.int32)
  call = _build_sc_call(b * s, hidden, jnp.bfloat16)
  out = call(ids, tok_emb, norm_weight)
  return out.reshape(b, s, hidden)
